# Initial kernel scaffold; baseline (speedup 1.0000x reference)
import jax
import jax.numpy as jnp
from jax.experimental import pallas as pl


def kernel(features, sample_idx):
    raise NotImplementedError("write your pallas kernel here")


# speedup vs baseline: 1.3007x; 1.3007x over previous
import functools

import jax
import jax.numpy as jnp
from jax import lax
from jax.experimental import pallas as pl
from jax.experimental.pallas import tpu as pltpu
from jax.experimental.pallas import tpu_sc as plsc

NC, NS = 2, 16
NW = NC * NS
CH = 128


def _gather_rows(flat, order):
  n, d = flat.shape
  r_per_w = n // NW
  g_chunks = r_per_w // CH
  mesh = plsc.VectorSubcoreMesh(core_axis_name="c", subcore_axis_name="s")

  @functools.partial(
      pl.kernel,
      out_type=jax.ShapeDtypeStruct((n, d), flat.dtype),
      mesh=mesh,
      scratch_types=[
          pltpu.VMEM((r_per_w,), jnp.int32),
          pltpu.VMEM((CH, d), flat.dtype),
          pltpu.SemaphoreType.DMA,
      ],
  )
  def gather_kernel(flat_hbm, order_hbm, out_hbm, idx_v, rows_v, sem0):
    wid = lax.axis_index("s") * NC + lax.axis_index("c")
    base = wid * r_per_w
    pltpu.sync_copy(order_hbm.at[pl.ds(base, r_per_w)], idx_v)

    def step(g, _):
      pltpu.async_copy(
          flat_hbm.at[idx_v.at[pl.ds(g * CH, CH)]], rows_v, sem0).wait()
      pltpu.sync_copy(rows_v, out_hbm.at[pl.ds(base + g * CH, CH), :])
      return 0

    lax.fori_loop(0, g_chunks, step, 0)

  return gather_kernel(flat, order)


def kernel(features, sample_idx):
  s, n_per, d = features.shape
  n = s * n_per
  flat = features.reshape(n, d)
  keys = sample_idx.reshape(n)
  order = jnp.argsort(keys)
  return _gather_rows(flat, order)


# speedup vs baseline: 1.4554x; 1.1189x over previous
import functools

import jax
import jax.numpy as jnp
from jax import lax
from jax.experimental import pallas as pl
from jax.experimental.pallas import tpu as pltpu
from jax.experimental.pallas import tpu_sc as plsc

NC, NS = 2, 16
NW = NC * NS
CH = 128


def _gather_rows(flat, order):
  n, d = flat.shape
  r_per_w = n // NW
  g_chunks = r_per_w // CH
  mesh = plsc.VectorSubcoreMesh(core_axis_name="c", subcore_axis_name="s")

  @functools.partial(
      pl.kernel,
      out_type=jax.ShapeDtypeStruct((n, d), flat.dtype),
      mesh=mesh,
      scratch_types=[
          pltpu.VMEM((r_per_w,), jnp.int32),
          pltpu.VMEM((CH, d), flat.dtype),
          pltpu.VMEM((CH, d), flat.dtype),
          pltpu.SemaphoreType.DMA,
          pltpu.SemaphoreType.DMA,
      ],
  )
  def gather_kernel(flat_hbm, order_hbm, out_hbm, idx_v, rows0, rows1,
                    sem0, sem1):
    wid = lax.axis_index("s") * NC + lax.axis_index("c")
    base = wid * r_per_w
    pltpu.sync_copy(order_hbm.at[pl.ds(base, r_per_w)], idx_v)
    bufs = ((rows0, sem0), (rows1, sem1))

    def fire(g, b):
      rows, sem = bufs[b]
      pltpu.async_copy(flat_hbm.at[idx_v.at[pl.ds(g * CH, CH)]], rows, sem)

    def wait_and_out(g, b):
      rows, sem = bufs[b]
      pltpu.make_async_copy(
          flat_hbm.at[idx_v.at[pl.ds(g * CH, CH)]], rows, sem).wait()
      pltpu.sync_copy(rows, out_hbm.at[pl.ds(base + g * CH, CH), :])

    fire(0, 0)
    fire(1, 1)

    def outer(g0, _):
      for b in range(2):
        g = g0 * 2 + b
        wait_and_out(g, b)
        fire(g + 2, b)
      return 0

    lax.fori_loop(0, g_chunks // 2 - 1, outer, 0)
    for b in range(2):
      wait_and_out(g_chunks - 2 + b, b)

  return gather_kernel(flat, order)


def kernel(features, sample_idx):
  s, n_per, d = features.shape
  n = s * n_per
  flat = features.reshape(n, d)
  keys = sample_idx.reshape(n)
  order = jnp.argsort(keys)
  return _gather_rows(flat, order)


# speedup vs baseline: 1.8428x; 1.2661x over previous
import functools

import jax
import jax.numpy as jnp
from jax import lax
from jax.experimental import pallas as pl
from jax.experimental.pallas import tpu as pltpu
from jax.experimental.pallas import tpu_sc as plsc

NC, NS = 2, 16
NW = NC * NS
CH = 128


def _sc_argsort(keys):
  n = keys.shape[0]
  nt = NS
  cpt = n // nt
  nv = cpt // 16
  nr = cpt // 128
  b1, b2 = 512, 512
  mesh = plsc.VectorSubcoreMesh(core_axis_name="c", subcore_axis_name="s")

  @functools.partial(
      pl.kernel,
      out_type=jax.ShapeDtypeStruct((n,), jnp.int32),
      mesh=mesh,
      compiler_params=pltpu.CompilerParams(needs_layout_passes=False),
      scratch_types=[
          pltpu.VMEM((cpt,), jnp.int32),
          pltpu.VMEM((cpt,), jnp.int32),
          pltpu.VMEM((nr, 128), jnp.int32),
          pltpu.VMEM((b1,), jnp.int32),
          pltpu.VMEM((nt, b1), jnp.int32),
          pltpu.VMEM_SHARED((nt, b1), jnp.int32),
          pltpu.VMEM_SHARED((n,), jnp.int32),
          pltpu.VMEM_SHARED((n,), jnp.int32),
          pltpu.VMEM_SHARED((n,), jnp.int32),
          pltpu.SemaphoreType.DMA,
      ],
  )
  def sort_kernel(keys_hbm, order_hbm, keys_v, vals_v, dest_v, off_v,
                  grid_v, grid_s, k1_s, j1_s, ord_s, sem):
    c = lax.axis_index("c")
    t = lax.axis_index("s")
    ones = jnp.full((16,), 1, jnp.int32)
    zeros = jnp.zeros((16,), jnp.int32)
    iota = lax.iota(jnp.int32, 16)

    def zero_off(nbins):
      def zloop(i, _):
        off_v[pl.ds(i * 16, 16)] = zeros
        return 0
      lax.fori_loop(0, nbins // 16, zloop, 0)

    def histogram(digit_fn):
      def hloop(i, _):
        d16 = digit_fn(keys_v[pl.ds(i * 16, 16)])
        plsc.addupdate_scatter(off_v, [d16], ones)
        return 0
      lax.fori_loop(0, nv, hloop, 0)

    def compute_offsets(nbins):
      def cloop(cb, carry):
        sl = pl.ds(cb * 16, 16)
        tot = zeros
        mine = zeros
        for tp in range(nt):
          row = grid_v[tp, sl]
          tot = tot + row
          mine = mine + row * (tp < t).astype(jnp.int32)
        cs = plsc.cumsum(tot)
        off_v[sl] = cs - tot + carry + mine
        return carry + jnp.sum(tot)
      lax.fori_loop(0, nbins // 16, cloop, jnp.int32(0))

    def rank_pass(digit_fn):
      def rloop(i, _):
        d16 = digit_fn(keys_v[pl.ds(i * 16, 16)])
        cnt16, _ = plsc.scan_count(d16)
        base16 = plsc.load_gather(off_v, [d16])
        dest_v[i // 8, pl.ds((i % 8) * 16, 16)] = base16 + cnt16 - 1
        plsc.addupdate_scatter(off_v, [d16], ones)
        return 0
      lax.fori_loop(0, nv, rloop, 0)

    def scatter_pairs(srcs_dsts):
      def floop(r, _):
        for src, dst in srcs_dsts:
          pltpu.async_copy(src.at[pl.ds(r * 128, 128)],
                           dst.at[dest_v.at[r]], sem)
        return 0
      lax.fori_loop(0, nr, floop, 0)
      def dloop(r, _):
        for src, dst in srcs_dsts:
          pltpu.make_async_copy(src.at[pl.ds(r * 128, 128)],
                                dst.at[dest_v.at[r]], sem).wait()
        return 0
      lax.fori_loop(0, nr, dloop, 0)

    @pl.when(c == 0)
    def _():
      base = t * cpt
      d1 = lambda k: lax.bitwise_and(k, jnp.int32(511))
      pltpu.sync_copy(keys_hbm.at[pl.ds(base, cpt)], keys_v)
      def vloop(i, _):
        vals_v[pl.ds(i * 16, 16)] = base + i * 16 + iota
        return 0
      lax.fori_loop(0, nv, vloop, 0)
      zero_off(b1)
      histogram(d1)
      pltpu.sync_copy(off_v, grid_s.at[t])
      plsc.subcore_barrier()
      pltpu.sync_copy(grid_s, grid_v)
      compute_offsets(b1)
      rank_pass(d1)
      scatter_pairs([(keys_v, k1_s), (vals_v, j1_s)])
      plsc.subcore_barrier()
      d2 = lambda k: lax.shift_right_logical(k, jnp.int32(9))
      pltpu.sync_copy(k1_s.at[pl.ds(base, cpt)], keys_v)
      pltpu.sync_copy(j1_s.at[pl.ds(base, cpt)], vals_v)
      zero_off(512)
      histogram(d2)
      pltpu.sync_copy(off_v, grid_s.at[t])
      plsc.subcore_barrier()
      pltpu.sync_copy(grid_s, grid_v)
      compute_offsets(256)
      rank_pass(d2)
      scatter_pairs([(vals_v, ord_s)])
      plsc.subcore_barrier()
      pltpu.sync_copy(ord_s.at[pl.ds(base, cpt)],
                      order_hbm.at[pl.ds(base, cpt)])

  return sort_kernel(keys)


def _gather_rows(flat, order):
  n, d = flat.shape
  r_per_w = n // NW
  g_chunks = r_per_w // CH
  mesh = plsc.VectorSubcoreMesh(core_axis_name="c", subcore_axis_name="s")

  @functools.partial(
      pl.kernel,
      out_type=jax.ShapeDtypeStruct((n, d), flat.dtype),
      mesh=mesh,
      scratch_types=[
          pltpu.VMEM((r_per_w,), jnp.int32),
          pltpu.VMEM((CH, d), flat.dtype),
          pltpu.VMEM((CH, d), flat.dtype),
          pltpu.SemaphoreType.DMA,
          pltpu.SemaphoreType.DMA,
      ],
  )
  def gather_kernel(flat_hbm, order_hbm, out_hbm, idx_v, rows0, rows1,
                    sem0, sem1):
    wid = lax.axis_index("s") * NC + lax.axis_index("c")
    base = wid * r_per_w
    pltpu.sync_copy(order_hbm.at[pl.ds(base, r_per_w)], idx_v)
    bufs = ((rows0, sem0), (rows1, sem1))

    def fire(g, b):
      rows, sem = bufs[b]
      pltpu.async_copy(flat_hbm.at[idx_v.at[pl.ds(g * CH, CH)]], rows, sem)

    def wait_and_out(g, b):
      rows, sem = bufs[b]
      pltpu.make_async_copy(
          flat_hbm.at[idx_v.at[pl.ds(g * CH, CH)]], rows, sem).wait()
      pltpu.sync_copy(rows, out_hbm.at[pl.ds(base + g * CH, CH), :])

    fire(0, 0)
    fire(1, 1)

    def outer(g0, _):
      for b in range(2):
        g = g0 * 2 + b
        wait_and_out(g, b)
        fire(g + 2, b)
      return 0

    lax.fori_loop(0, g_chunks // 2 - 1, outer, 0)
    for b in range(2):
      wait_and_out(g_chunks - 2 + b, b)

  return gather_kernel(flat, order)


def kernel(features, sample_idx):
  s, n_per, d = features.shape
  n = s * n_per
  flat = features.reshape(n, d)
  keys = sample_idx.reshape(n)
  order = _sc_argsort(keys)
  return _gather_rows(flat, order)


# speedup vs baseline: 1.8934x; 1.0275x over previous
import functools

import jax
import jax.numpy as jnp
from jax import lax
from jax.experimental import pallas as pl
from jax.experimental.pallas import tpu as pltpu
from jax.experimental.pallas import tpu_sc as plsc

NC, NS = 2, 16
NW = NC * NS
CH = 128


def _densify(flat, keys):
  n, d = flat.shape
  nt = NS
  cpt = n // nt
  nv = cpt // 16
  nr = cpt // 128
  b1 = 512
  r_per_w = n // NW
  g_chunks = r_per_w // CH
  mesh = plsc.VectorSubcoreMesh(core_axis_name="c", subcore_axis_name="s")

  @functools.partial(
      pl.kernel,
      out_type=jax.ShapeDtypeStruct((n, d), flat.dtype),
      mesh=mesh,
      compiler_params=pltpu.CompilerParams(needs_layout_passes=False),
      scratch_types=[
          pltpu.VMEM((cpt,), jnp.int32),
          pltpu.VMEM((cpt,), jnp.int32),
          pltpu.VMEM((nr, 128), jnp.int32),
          pltpu.VMEM((b1,), jnp.int32),
          pltpu.VMEM((nt, b1), jnp.int32),
          pltpu.VMEM((r_per_w,), jnp.int32),
          pltpu.VMEM((CH, d), jnp.float32),
          pltpu.VMEM((CH, d), jnp.float32),
          pltpu.VMEM_SHARED((nt, b1), jnp.int32),
          pltpu.VMEM_SHARED((n,), jnp.int32),
          pltpu.VMEM_SHARED((n,), jnp.int32),
          pltpu.VMEM_SHARED((n,), jnp.int32),
          pltpu.SemaphoreType.DMA,
          pltpu.SemaphoreType.DMA,
          pltpu.SemaphoreType.DMA,
      ],
  )
  def densify_kernel(flat_hbm, keys_hbm, out_hbm, keys_v, vals_v, dest_v,
                     off_v, grid_v, idx_v, rows0, rows1, grid_s, k1_s, j1_s,
                     ord_s, sem, sem0, sem1):
    c = lax.axis_index("c")
    t = lax.axis_index("s")
    ones = jnp.full((16,), 1, jnp.int32)
    zeros = jnp.zeros((16,), jnp.int32)
    iota = lax.iota(jnp.int32, 16)
    base = t * cpt

    def zero_off(nbins):
      def zloop(i, _):
        off_v[pl.ds(i * 16, 16)] = zeros
        return 0
      lax.fori_loop(0, nbins // 16, zloop, 0)

    def histogram(digit_fn):
      def hloop(i, _):
        d16 = digit_fn(keys_v[pl.ds(i * 16, 16)])
        plsc.addupdate_scatter(off_v, [d16], ones)
        return 0
      lax.fori_loop(0, nv, hloop, 0)

    def compute_offsets(nbins):
      def cloop(cb, carry):
        sl = pl.ds(cb * 16, 16)
        tot = zeros
        mine = zeros
        for tp in range(nt):
          row = grid_v[tp, sl]
          tot = tot + row
          mine = mine + row * (tp < t).astype(jnp.int32)
        cs = plsc.cumsum(tot)
        off_v[sl] = cs - tot + carry + mine
        return carry + jnp.sum(tot)
      lax.fori_loop(0, nbins // 16, cloop, jnp.int32(0))

    def rank_pass(digit_fn):
      def rloop(i, _):
        d16 = digit_fn(keys_v[pl.ds(i * 16, 16)])
        cnt16, _ = plsc.scan_count(d16)
        base16 = plsc.load_gather(off_v, [d16])
        dest_v[i // 8, pl.ds((i % 8) * 16, 16)] = base16 + cnt16 - 1
        plsc.addupdate_scatter(off_v, [d16], ones)
        return 0
      lax.fori_loop(0, nv, rloop, 0)

    def scatter_pairs(srcs_dsts):
      def floop(r, _):
        for src, dst in srcs_dsts:
          pltpu.async_copy(src.at[pl.ds(r * 128, 128)],
                           dst.at[dest_v.at[r]], sem)
        return 0
      lax.fori_loop(0, nr, floop, 0)
      def dloop(r, _):
        for src, dst in srcs_dsts:
          pltpu.make_async_copy(src.at[pl.ds(r * 128, 128)],
                                dst.at[dest_v.at[r]], sem).wait()
        return 0
      lax.fori_loop(0, nr, dloop, 0)

    d1 = lambda k: lax.bitwise_and(k, jnp.int32(511))
    pltpu.sync_copy(keys_hbm.at[pl.ds(base, cpt)], keys_v)
    def vloop(i, _):
      vals_v[pl.ds(i * 16, 16)] = base + i * 16 + iota
      return 0
    lax.fori_loop(0, nv, vloop, 0)
    zero_off(b1)
    histogram(d1)
    pltpu.sync_copy(off_v, grid_s.at[t])
    plsc.subcore_barrier()
    pltpu.sync_copy(grid_s, grid_v)
    compute_offsets(b1)
    rank_pass(d1)
    scatter_pairs([(keys_v, k1_s), (vals_v, j1_s)])
    plsc.subcore_barrier()
    d2 = lambda k: lax.shift_right_logical(k, jnp.int32(9))
    pltpu.sync_copy(k1_s.at[pl.ds(base, cpt)], keys_v)
    pltpu.sync_copy(j1_s.at[pl.ds(base, cpt)], vals_v)
    zero_off(512)
    histogram(d2)
    pltpu.sync_copy(off_v, grid_s.at[t])
    plsc.subcore_barrier()
    pltpu.sync_copy(grid_s, grid_v)
    compute_offsets(256)
    rank_pass(d2)
    scatter_pairs([(vals_v, ord_s)])
    plsc.subcore_barrier()

    wid = c * NS + t
    wbase = wid * r_per_w
    pltpu.sync_copy(ord_s.at[pl.ds(wbase, r_per_w)], idx_v)
    bufs = ((rows0, sem0), (rows1, sem1))

    def fire(g, b):
      rows, gsem = bufs[b]
      pltpu.async_copy(flat_hbm.at[idx_v.at[pl.ds(g * CH, CH)]], rows, gsem)

    def wait_and_out(g, b):
      rows, gsem = bufs[b]
      pltpu.make_async_copy(
          flat_hbm.at[idx_v.at[pl.ds(g * CH, CH)]], rows, gsem).wait()
      pltpu.sync_copy(rows, out_hbm.at[pl.ds(wbase + g * CH, CH), :])

    fire(0, 0)
    fire(1, 1)

    def outer(g0, _):
      for b in range(2):
        g = g0 * 2 + b
        wait_and_out(g, b)
        fire(g + 2, b)
      return 0

    lax.fori_loop(0, g_chunks // 2 - 1, outer, 0)
    for b in range(2):
      wait_and_out(g_chunks - 2 + b, b)

  return densify_kernel(flat, keys)


def kernel(features, sample_idx):
  s, n_per, d = features.shape
  n = s * n_per
  flat = features.reshape(n, d)
  keys = sample_idx.reshape(n)
  return _densify(flat, keys)


# speedup vs baseline: 1.9265x; 1.0175x over previous
import functools

import jax
import jax.numpy as jnp
from jax import lax
from jax.experimental import pallas as pl
from jax.experimental.pallas import tpu as pltpu
from jax.experimental.pallas import tpu_sc as plsc

NC, NS = 2, 16
NW = NC * NS
CH = 128


def _densify(flat, keys):
  n, d = flat.shape
  nt = NS
  nu = 4
  cpt = n // nt
  cpu_ = cpt // nu
  ng = cpu_ // 16
  nr = cpt // 128
  b1 = 512
  r_per_w = n // NW
  g_chunks = r_per_w // CH
  mesh = plsc.VectorSubcoreMesh(core_axis_name="c", subcore_axis_name="s")

  @functools.partial(
      pl.kernel,
      out_type=jax.ShapeDtypeStruct((n, d), flat.dtype),
      mesh=mesh,
      compiler_params=pltpu.CompilerParams(needs_layout_passes=False),
      scratch_types=[
          pltpu.VMEM((cpt,), jnp.int32),
          pltpu.VMEM((cpt,), jnp.int32),
          pltpu.VMEM((nr, 128), jnp.int32),
          pltpu.VMEM((nu * b1,), jnp.int32),
          pltpu.VMEM((b1,), jnp.int32),
          pltpu.VMEM((nt, b1), jnp.int32),
          pltpu.VMEM((r_per_w,), jnp.int32),
          pltpu.VMEM((CH, d), jnp.float32),
          pltpu.VMEM((CH, d), jnp.float32),
          pltpu.VMEM_SHARED((nt, b1), jnp.int32),
          pltpu.VMEM_SHARED((n,), jnp.int32),
          pltpu.VMEM_SHARED((n,), jnp.int32),
          pltpu.SemaphoreType.DMA,
          pltpu.SemaphoreType.DMA,
          pltpu.SemaphoreType.DMA,
      ],
  )
  def densify_kernel(flat_hbm, keys_hbm, out_hbm, keys_v, vals_v, dest_v,
                     off_v, tsum_v, grid_v, idx_v, rows0, rows1, grid_s,
                     p1_s, ord_s, sem, sem0, sem1):
    c = lax.axis_index("c")
    t = lax.axis_index("s")
    ones = jnp.full((16,), 1, jnp.int32)
    zeros = jnp.zeros((16,), jnp.int32)
    iota = lax.iota(jnp.int32, 16)
    base = t * cpt

    def zero_off():
      def zloop(i, _):
        off_v[pl.ds(i * 16, 16)] = zeros
        return 0
      lax.fori_loop(0, nu * b1 // 16, zloop, 0)

    def histogram(digit_fn):
      def hloop(i, _):
        for u in range(nu):
          d16 = digit_fn(keys_v[pl.ds(u * cpu_ + i * 16, 16)])
          plsc.addupdate_scatter(off_v, [d16 + u * b1], ones)
        return 0
      lax.fori_loop(0, ng, hloop, 0)

    def tile_totals(nbins):
      def tloop(cb, _):
        sl = pl.ds(cb * 16, 16)
        tot = zeros
        for u in range(nu):
          tot = tot + off_v[pl.ds(u * b1 + cb * 16, 16)]
        tsum_v[sl] = tot
        return 0
      lax.fori_loop(0, nbins // 16, tloop, 0)

    def compute_offsets(nbins):
      def cloop(cb, carry):
        sl = pl.ds(cb * 16, 16)
        tot = zeros
        mine = zeros
        for tp in range(nt):
          row = grid_v[tp, sl]
          tot = tot + row
          mine = mine + row * (tp < t).astype(jnp.int32)
        cs = plsc.cumsum(tot)
        run = cs - tot + carry + mine
        for u in range(nu):
          h = off_v[pl.ds(u * b1 + cb * 16, 16)]
          off_v[pl.ds(u * b1 + cb * 16, 16)] = run
          run = run + h
        return carry + jnp.sum(tot)
      lax.fori_loop(0, nbins // 16, cloop, jnp.int32(0))

    def rank_pass(digit_fn, val_fn):
      def rloop(i, _):
        for u in range(nu):
          pos = u * cpu_ + i * 16
          k16 = keys_v[pl.ds(pos, 16)]
          d16 = digit_fn(k16)
          cnt16, _ = plsc.scan_count(d16)
          base16 = plsc.load_gather(off_v, [d16 + u * b1])
          row = u * (cpu_ // 128) + i // 8
          dest_v[row, pl.ds((i % 8) * 16, 16)] = base16 + cnt16 - 1
          plsc.addupdate_scatter(off_v, [d16 + u * b1], ones)
          vals_v[pl.ds(pos, 16)] = val_fn(k16, base + pos + iota)
        return 0
      lax.fori_loop(0, ng, rloop, 0)

    def scatter_vals(dst):
      def floop(r, _):
        pltpu.async_copy(vals_v.at[pl.ds(r * 128, 128)],
                         dst.at[dest_v.at[r]], sem)
        return 0
      lax.fori_loop(0, nr, floop, 0)
      def dloop(r, _):
        pltpu.make_async_copy(vals_v.at[pl.ds(r * 128, 128)],
                              dst.at[dest_v.at[r]], sem).wait()
        return 0
      lax.fori_loop(0, nr, dloop, 0)

    d1 = lambda k: lax.bitwise_and(k, jnp.int32(511))
    pack1 = lambda k, j: lax.bitwise_or(
        lax.shift_left(lax.shift_right_logical(k, jnp.int32(9)),
                       jnp.int32(20)), j)
    pltpu.sync_copy(keys_hbm.at[pl.ds(base, cpt)], keys_v)
    zero_off()
    histogram(d1)
    tile_totals(b1)
    pltpu.sync_copy(tsum_v, grid_s.at[t])
    plsc.subcore_barrier()
    pltpu.sync_copy(grid_s, grid_v)
    compute_offsets(b1)
    rank_pass(d1, pack1)
    scatter_vals(p1_s)
    plsc.subcore_barrier()
    d2 = lambda p: lax.shift_right_logical(p, jnp.int32(20))
    unpack_j = lambda p, _: lax.bitwise_and(p, jnp.int32((1 << 20) - 1))
    pltpu.sync_copy(p1_s.at[pl.ds(base, cpt)], keys_v)
    zero_off()
    histogram(d2)
    tile_totals(256)
    pltpu.sync_copy(tsum_v, grid_s.at[t])
    plsc.subcore_barrier()
    pltpu.sync_copy(grid_s, grid_v)
    compute_offsets(256)
    rank_pass(d2, unpack_j)
    scatter_vals(ord_s)
    plsc.subcore_barrier()

    wid = c * NS + t
    wbase = wid * r_per_w
    pltpu.sync_copy(ord_s.at[pl.ds(wbase, r_per_w)], idx_v)
    bufs = ((rows0, sem0), (rows1, sem1))

    def fire(g, b):
      rows, gsem = bufs[b]
      pltpu.async_copy(flat_hbm.at[idx_v.at[pl.ds(g * CH, CH)]], rows, gsem)

    def wait_and_out(g, b):
      rows, gsem = bufs[b]
      pltpu.make_async_copy(
          flat_hbm.at[idx_v.at[pl.ds(g * CH, CH)]], rows, gsem).wait()
      pltpu.sync_copy(rows, out_hbm.at[pl.ds(wbase + g * CH, CH), :])

    fire(0, 0)
    fire(1, 1)

    def outer(g0, _):
      for b in range(2):
        g = g0 * 2 + b
        wait_and_out(g, b)
        fire(g + 2, b)
      return 0

    lax.fori_loop(0, g_chunks // 2 - 1, outer, 0)
    for b in range(2):
      wait_and_out(g_chunks - 2 + b, b)

  return densify_kernel(flat, keys)


def kernel(features, sample_idx):
  s, n_per, d = features.shape
  n = s * n_per
  flat = features.reshape(n, d)
  keys = sample_idx.reshape(n)
  return _densify(flat, keys)


# speedup vs baseline: 2.0511x; 1.0647x over previous
import functools

import jax
import jax.numpy as jnp
from jax import lax
from jax.experimental import pallas as pl
from jax.experimental.pallas import tpu as pltpu
from jax.experimental.pallas import tpu_sc as plsc

NC, NS = 2, 16
NW = NC * NS
CH = 128


def _densify(flat, keys):
  n, d = flat.shape
  nt = NS
  nu = 4
  cpt = n // nt
  cpu_ = cpt // nu
  ng = cpu_ // 16
  nr = cpt // 128
  b1 = 512
  r_per_w = n // NW
  g_chunks = r_per_w // CH
  mesh = plsc.VectorSubcoreMesh(core_axis_name="c", subcore_axis_name="s")

  @functools.partial(
      pl.kernel,
      out_type=jax.ShapeDtypeStruct((n, d), flat.dtype),
      mesh=mesh,
      compiler_params=pltpu.CompilerParams(needs_layout_passes=False),
      scratch_types=[
          pltpu.VMEM((cpt,), jnp.int32),
          pltpu.VMEM((cpt,), jnp.int32),
          pltpu.VMEM((cpt,), jnp.int32),
          pltpu.VMEM((nr, 128), jnp.int32),
          [pltpu.VMEM((b1,), jnp.int32)] * nu,
          pltpu.VMEM((b1,), jnp.int32),
          pltpu.VMEM((nt, b1), jnp.int32),
          pltpu.VMEM((r_per_w,), jnp.int32),
          pltpu.VMEM((CH, d), jnp.float32),
          pltpu.VMEM((CH, d), jnp.float32),
          pltpu.VMEM_SHARED((nt, b1), jnp.int32),
          pltpu.VMEM_SHARED((n,), jnp.int32),
          pltpu.VMEM_SHARED((n,), jnp.int32),
          pltpu.SemaphoreType.DMA,
          pltpu.SemaphoreType.DMA,
          pltpu.SemaphoreType.DMA,
      ],
  )
  def densify_kernel(flat_hbm, keys_hbm, out_hbm, keys_v, vals_v, occ_v,
                     dest_v, offs, tsum_v, grid_v, idx_v, rows0, rows1,
                     grid_s, p1_s, ord_s, sem, sem0, sem1):
    c = lax.axis_index("c")
    t = lax.axis_index("s")
    ones = jnp.full((16,), 1, jnp.int32)
    zeros = jnp.zeros((16,), jnp.int32)
    iota = lax.iota(jnp.int32, 16)
    base = t * cpt

    def zero_off():
      @plsc.parallel_loop(0, b1 // 16)
      def zloop(i):
        for u in range(nu):
          offs[u][pl.ds(i * 16, 16)] = zeros

    def histogram(digit_fn, val_fn):
      @plsc.parallel_loop(0, ng, unroll=2)
      def hloop(i):
        for u in range(nu):
          pos = u * cpu_ + i * 16
          k16 = keys_v[pl.ds(pos, 16)]
          d16 = digit_fn(k16)
          cnt16, _ = plsc.scan_count(d16)
          occ_v[pl.ds(pos, 16)] = cnt16
          vals_v[pl.ds(pos, 16)] = val_fn(k16, base + pos + iota)
          plsc.addupdate_scatter(offs[u], [d16], ones)

    def tile_totals(nbins):
      @plsc.parallel_loop(0, nbins // 16, unroll=2)
      def tloop(cb):
        sl = pl.ds(cb * 16, 16)
        tot = zeros
        for u in range(nu):
          tot = tot + offs[u][pl.ds(cb * 16, 16)]
        tsum_v[sl] = tot

    def compute_offsets(nbins):
      @plsc.parallel_loop(0, nbins // 16, carry=jnp.int32(0))
      def cloop(cb, carry):
        sl = pl.ds(cb * 16, 16)
        tot = zeros
        mine = zeros
        for tp in range(nt):
          row = grid_v[tp, sl]
          tot = tot + row
          mine = mine + row * (tp < t).astype(jnp.int32)
        cs = plsc.cumsum(tot)
        run = cs - tot + carry + mine
        for u in range(nu):
          h = offs[u][pl.ds(cb * 16, 16)]
          offs[u][pl.ds(cb * 16, 16)] = run
          run = run + h
        return carry + jnp.sum(tot)

    def rank_pass(digit_fn):
      def rloop(i, _):
        for u in range(nu):
          pos = u * cpu_ + i * 16
          d16 = digit_fn(keys_v[pl.ds(pos, 16)])
          base16 = plsc.load_gather(offs[u], [d16])
          row = u * (cpu_ // 128) + i // 8
          dest_v[row, pl.ds((i % 8) * 16, 16)] = (
              base16 + occ_v[pl.ds(pos, 16)] - 1)
          plsc.addupdate_scatter(offs[u], [d16], ones)
        return 0
      lax.fori_loop(0, ng, rloop, 0)

    def scatter_vals(dst):
      def floop(r, _):
        pltpu.async_copy(vals_v.at[pl.ds(r * 128, 128)],
                         dst.at[dest_v.at[r]], sem)
        return 0
      lax.fori_loop(0, nr, floop, 0)
      def dloop(r, _):
        pltpu.make_async_copy(vals_v.at[pl.ds(r * 128, 128)],
                              dst.at[dest_v.at[r]], sem).wait()
        return 0
      lax.fori_loop(0, nr, dloop, 0)

    d1 = lambda k: lax.bitwise_and(k, jnp.int32(511))
    pack1 = lambda k, j: lax.bitwise_or(
        lax.shift_left(lax.shift_right_logical(k, jnp.int32(9)),
                       jnp.int32(20)), j)
    pltpu.sync_copy(keys_hbm.at[pl.ds(base, cpt)], keys_v)
    zero_off()
    histogram(d1, pack1)
    tile_totals(b1)
    pltpu.sync_copy(tsum_v, grid_s.at[t])
    plsc.subcore_barrier()
    pltpu.sync_copy(grid_s, grid_v)
    compute_offsets(b1)
    rank_pass(d1)
    scatter_vals(p1_s)
    plsc.subcore_barrier()
    d2 = lambda p: lax.shift_right_logical(p, jnp.int32(20))
    unpack_j = lambda p, _: lax.bitwise_and(p, jnp.int32((1 << 20) - 1))
    pltpu.sync_copy(p1_s.at[pl.ds(base, cpt)], keys_v)
    zero_off()
    histogram(d2, unpack_j)
    tile_totals(256)
    pltpu.sync_copy(tsum_v, grid_s.at[t])
    plsc.subcore_barrier()
    pltpu.sync_copy(grid_s, grid_v)
    compute_offsets(256)
    rank_pass(d2)
    scatter_vals(ord_s)
    plsc.subcore_barrier()

    wid = c * NS + t
    wbase = wid * r_per_w
    pltpu.sync_copy(ord_s.at[pl.ds(wbase, r_per_w)], idx_v)
    bufs = ((rows0, sem0), (rows1, sem1))

    def fire(g, b):
      rows, gsem = bufs[b]
      pltpu.async_copy(flat_hbm.at[idx_v.at[pl.ds(g * CH, CH)]], rows, gsem)

    def wait_and_out(g, b):
      rows, gsem = bufs[b]
      pltpu.make_async_copy(
          flat_hbm.at[idx_v.at[pl.ds(g * CH, CH)]], rows, gsem).wait()
      pltpu.sync_copy(rows, out_hbm.at[pl.ds(wbase + g * CH, CH), :])

    fire(0, 0)
    fire(1, 1)

    def outer(g0, _):
      for b in range(2):
        g = g0 * 2 + b
        wait_and_out(g, b)
        fire(g + 2, b)
      return 0

    lax.fori_loop(0, g_chunks // 2 - 1, outer, 0)
    for b in range(2):
      wait_and_out(g_chunks - 2 + b, b)

  return densify_kernel(flat, keys)


def kernel(features, sample_idx):
  s, n_per, d = features.shape
  n = s * n_per
  flat = features.reshape(n, d)
  keys = sample_idx.reshape(n)
  return _densify(flat, keys)


# speedup vs baseline: 2.0659x; 1.0072x over previous
import functools

import jax
import jax.numpy as jnp
from jax import lax
from jax.experimental import pallas as pl
from jax.experimental.pallas import tpu as pltpu
from jax.experimental.pallas import tpu_sc as plsc

NC, NS = 2, 16
NW = NC * NS
CH = 64
NB = 4


def _densify(flat, keys):
  n, d = flat.shape
  nt = NS
  nu = 4
  cpt = n // nt
  cpu_ = cpt // nu
  ng = cpu_ // 16
  nr = cpt // 128
  b1 = 512
  r_per_w = n // NW
  g_chunks = r_per_w // CH
  mesh = plsc.VectorSubcoreMesh(core_axis_name="c", subcore_axis_name="s")

  @functools.partial(
      pl.kernel,
      out_type=jax.ShapeDtypeStruct((n, d), flat.dtype),
      mesh=mesh,
      compiler_params=pltpu.CompilerParams(needs_layout_passes=False),
      scratch_types=[
          pltpu.VMEM((cpt,), jnp.int32),
          pltpu.VMEM((cpt,), jnp.int32),
          pltpu.VMEM((cpt,), jnp.int32),
          pltpu.VMEM((nr, 128), jnp.int32),
          [pltpu.VMEM((b1,), jnp.int32)] * nu,
          pltpu.VMEM((b1,), jnp.int32),
          pltpu.VMEM((nt, b1), jnp.int32),
          pltpu.VMEM((r_per_w,), jnp.int32),
          [pltpu.VMEM((CH, d), jnp.float32)] * NB,
          pltpu.VMEM_SHARED((nt, b1), jnp.int32),
          pltpu.VMEM_SHARED((n,), jnp.int32),
          pltpu.VMEM_SHARED((n,), jnp.int32),
          pltpu.SemaphoreType.DMA,
          [pltpu.SemaphoreType.DMA] * NB,
          [pltpu.SemaphoreType.DMA] * NB,
      ],
  )
  def densify_kernel(flat_hbm, keys_hbm, out_hbm, keys_v, vals_v, occ_v,
                     dest_v, offs, tsum_v, grid_v, idx_v, bufs,
                     grid_s, p1_s, ord_s, sem, gsems, wsems):
    c = lax.axis_index("c")
    t = lax.axis_index("s")
    ones = jnp.full((16,), 1, jnp.int32)
    zeros = jnp.zeros((16,), jnp.int32)
    iota = lax.iota(jnp.int32, 16)
    base = t * cpt

    def zero_off():
      @plsc.parallel_loop(0, b1 // 16)
      def zloop(i):
        for u in range(nu):
          offs[u][pl.ds(i * 16, 16)] = zeros

    def histogram(digit_fn, val_fn):
      @plsc.parallel_loop(0, ng, unroll=2)
      def hloop(i):
        for u in range(nu):
          pos = u * cpu_ + i * 16
          k16 = keys_v[pl.ds(pos, 16)]
          d16 = digit_fn(k16)
          cnt16, _ = plsc.scan_count(d16)
          occ_v[pl.ds(pos, 16)] = cnt16
          vals_v[pl.ds(pos, 16)] = val_fn(k16, base + pos + iota)
          plsc.addupdate_scatter(offs[u], [d16], ones)

    def tile_totals(nbins):
      @plsc.parallel_loop(0, nbins // 16, unroll=2)
      def tloop(cb):
        sl = pl.ds(cb * 16, 16)
        tot = zeros
        for u in range(nu):
          tot = tot + offs[u][pl.ds(cb * 16, 16)]
        tsum_v[sl] = tot

    def compute_offsets(nbins):
      @plsc.parallel_loop(0, nbins // 16, carry=jnp.int32(0))
      def cloop(cb, carry):
        sl = pl.ds(cb * 16, 16)
        tot = zeros
        mine = zeros
        for tp in range(nt):
          row = grid_v[tp, sl]
          tot = tot + row
          mine = mine + row * (tp < t).astype(jnp.int32)
        cs = plsc.cumsum(tot)
        run = cs - tot + carry + mine
        for u in range(nu):
          h = offs[u][pl.ds(cb * 16, 16)]
          offs[u][pl.ds(cb * 16, 16)] = run
          run = run + h
        return carry + jnp.sum(tot)

    def rank_pass(digit_fn):
      def rloop(i, _):
        for u in range(nu):
          pos = u * cpu_ + i * 16
          d16 = digit_fn(keys_v[pl.ds(pos, 16)])
          base16 = plsc.load_gather(offs[u], [d16])
          row = u * (cpu_ // 128) + i // 8
          dest_v[row, pl.ds((i % 8) * 16, 16)] = (
              base16 + occ_v[pl.ds(pos, 16)] - 1)
          plsc.addupdate_scatter(offs[u], [d16], ones)
        return 0
      lax.fori_loop(0, ng, rloop, 0)

    def scatter_vals(dst):
      def floop(r, _):
        pltpu.async_copy(vals_v.at[pl.ds(r * 128, 128)],
                         dst.at[dest_v.at[r]], sem)
        return 0
      lax.fori_loop(0, nr, floop, 0)
      def dloop(r, _):
        pltpu.make_async_copy(vals_v.at[pl.ds(r * 128, 128)],
                              dst.at[dest_v.at[r]], sem).wait()
        return 0
      lax.fori_loop(0, nr, dloop, 0)

    d1 = lambda k: lax.bitwise_and(k, jnp.int32(511))
    pack1 = lambda k, j: lax.bitwise_or(
        lax.shift_left(lax.shift_right_logical(k, jnp.int32(9)),
                       jnp.int32(20)), j)
    pltpu.sync_copy(keys_hbm.at[pl.ds(base, cpt)], keys_v)
    zero_off()
    histogram(d1, pack1)
    tile_totals(b1)
    pltpu.sync_copy(tsum_v, grid_s.at[t])
    plsc.subcore_barrier()
    pltpu.sync_copy(grid_s, grid_v)
    compute_offsets(b1)
    rank_pass(d1)
    scatter_vals(p1_s)
    plsc.subcore_barrier()
    d2 = lambda p: lax.shift_right_logical(p, jnp.int32(20))
    unpack_j = lambda p, _: lax.bitwise_and(p, jnp.int32((1 << 20) - 1))
    pltpu.sync_copy(p1_s.at[pl.ds(base, cpt)], keys_v)
    zero_off()
    histogram(d2, unpack_j)
    tile_totals(256)
    pltpu.sync_copy(tsum_v, grid_s.at[t])
    plsc.subcore_barrier()
    pltpu.sync_copy(grid_s, grid_v)
    compute_offsets(256)
    rank_pass(d2)
    scatter_vals(ord_s)
    plsc.subcore_barrier()

    wid = c * NS + t
    wbase = wid * r_per_w
    pltpu.sync_copy(ord_s.at[pl.ds(wbase, r_per_w)], idx_v)

    def fire_gather(g, b):
      pltpu.async_copy(flat_hbm.at[idx_v.at[pl.ds(g * CH, CH)]], bufs[b],
                       gsems[b])

    def wait_gather(g, b):
      pltpu.make_async_copy(flat_hbm.at[idx_v.at[pl.ds(g * CH, CH)]], bufs[b],
                            gsems[b]).wait()

    def fire_write(g, b):
      pltpu.async_copy(bufs[b], out_hbm.at[pl.ds(wbase + g * CH, CH), :],
                       wsems[b])

    def wait_write(g, b):
      pltpu.make_async_copy(bufs[b], out_hbm.at[pl.ds(wbase + g * CH, CH), :],
                            wsems[b]).wait()

    fire_gather(0, 0)
    fire_gather(1, 1)
    for g in (0, 1):
      wait_gather(g, g)
      fire_write(g, g)
      fire_gather(g + 2, g + 2)

    def outer(j, _):
      for k in range(4):
        g = 2 + j * 4 + k
        b = (2 + k) % 4
        b2 = (b + 2) % 4
        wait_gather(g, b)
        fire_write(g, b)
        wait_write(g - 2, b2)
        fire_gather(g + 2, b2)
      return 0

    lax.fori_loop(0, (g_chunks - 4) // 4, outer, 0)
    for g in (g_chunks - 2, g_chunks - 1):
      wait_gather(g, g % 4)
      fire_write(g, g % 4)
    for g in range(g_chunks - 4, g_chunks):
      wait_write(g, g % 4)

  return densify_kernel(flat, keys)


def kernel(features, sample_idx):
  s, n_per, d = features.shape
  n = s * n_per
  flat = features.reshape(n, d)
  keys = sample_idx.reshape(n)
  return _densify(flat, keys)


# speedup vs baseline: 2.0817x; 1.0077x over previous
import functools

import jax
import jax.numpy as jnp
from jax import lax
from jax.experimental import pallas as pl
from jax.experimental.pallas import tpu as pltpu
from jax.experimental.pallas import tpu_sc as plsc

NC, NS = 2, 16
NW = NC * NS
CH = 64
NB = 4


def _densify(flat, keys):
  n, d = flat.shape
  nt = NS
  nu = 4
  cpt = n // nt
  cpu_ = cpt // nu
  ng = cpu_ // 16
  nr = cpt // 128
  b1 = 512
  r_per_w = n // NW
  g_chunks = r_per_w // CH
  mesh = plsc.VectorSubcoreMesh(core_axis_name="c", subcore_axis_name="s")

  @functools.partial(
      pl.kernel,
      out_type=jax.ShapeDtypeStruct((n, d), flat.dtype),
      mesh=mesh,
      compiler_params=pltpu.CompilerParams(needs_layout_passes=False),
      scratch_types=[
          pltpu.VMEM((cpt,), jnp.int32),
          pltpu.VMEM((cpt,), jnp.int32),
          pltpu.VMEM((cpt,), jnp.int32),
          pltpu.VMEM((nr, 128), jnp.int32),
          [pltpu.VMEM((b1,), jnp.int32)] * nu,
          pltpu.VMEM((b1,), jnp.int32),
          pltpu.VMEM((nt, b1), jnp.int32),
          pltpu.VMEM((r_per_w,), jnp.int32),
          [pltpu.VMEM((CH, d), jnp.float32)] * NB,
          pltpu.VMEM_SHARED((nt, b1), jnp.int32),
          pltpu.VMEM_SHARED((n,), jnp.int32),
          pltpu.VMEM_SHARED((n,), jnp.int32),
          pltpu.SemaphoreType.DMA,
          [pltpu.SemaphoreType.DMA] * NB,
          [pltpu.SemaphoreType.DMA] * NB,
      ],
  )
  def densify_kernel(flat_hbm, keys_hbm, out_hbm, keys_v, vals_v, occ_v,
                     dest_v, offs, tsum_v, grid_v, idx_v, bufs,
                     grid_s, p1_s, ord_s, sem, gsems, wsems):
    c = lax.axis_index("c")
    t = lax.axis_index("s")
    ones = jnp.full((16,), 1, jnp.int32)
    zeros = jnp.zeros((16,), jnp.int32)
    iota = lax.iota(jnp.int32, 16)
    base = t * cpt

    def zero_off():
      @plsc.parallel_loop(0, b1 // 16)
      def zloop(i):
        for u in range(nu):
          offs[u][pl.ds(i * 16, 16)] = zeros

    def histogram(digit_fn, val_fn):
      @plsc.parallel_loop(0, ng, unroll=2)
      def hloop(i):
        for u in range(nu):
          pos = u * cpu_ + i * 16
          k16 = keys_v[pl.ds(pos, 16)]
          d16 = digit_fn(k16)
          cnt16, _ = plsc.scan_count(d16)
          occ_v[pl.ds(pos, 16)] = cnt16 - 1
          vals_v[pl.ds(pos, 16)] = val_fn(k16, base + pos + iota)
          keys_v[pl.ds(pos, 16)] = d16
          plsc.addupdate_scatter(offs[u], [d16], ones)

    def tile_totals(nbins):
      @plsc.parallel_loop(0, nbins // 16, unroll=2)
      def tloop(cb):
        sl = pl.ds(cb * 16, 16)
        tot = zeros
        for u in range(nu):
          tot = tot + offs[u][pl.ds(cb * 16, 16)]
        tsum_v[sl] = tot

    def compute_offsets(nbins):
      @plsc.parallel_loop(0, nbins // 16, carry=jnp.int32(0))
      def cloop(cb, carry):
        sl = pl.ds(cb * 16, 16)
        tot = zeros
        mine = zeros
        for tp in range(nt):
          row = grid_v[tp, sl]
          tot = tot + row
          mine = mine + row * (tp < t).astype(jnp.int32)
        cs = plsc.cumsum(tot)
        run = cs - tot + carry + mine
        for u in range(nu):
          h = offs[u][pl.ds(cb * 16, 16)]
          offs[u][pl.ds(cb * 16, 16)] = run
          run = run + h
        return carry + jnp.sum(tot)

    def rank_pass(digit_fn):
      def rloop(i, _):
        for u in range(nu):
          pos = u * cpu_ + i * 16
          d16 = keys_v[pl.ds(pos, 16)]
          base16 = plsc.load_gather(offs[u], [d16])
          row = u * (cpu_ // 128) + i // 8
          dest_v[row, pl.ds((i % 8) * 16, 16)] = (
              base16 + occ_v[pl.ds(pos, 16)])
          plsc.addupdate_scatter(offs[u], [d16], ones)
        return 0
      lax.fori_loop(0, ng, rloop, 0)

    def scatter_vals(dst):
      def floop(r, _):
        pltpu.async_copy(vals_v.at[pl.ds(r * 128, 128)],
                         dst.at[dest_v.at[r]], sem)
        return 0
      lax.fori_loop(0, nr, floop, 0)
      def dloop(r, _):
        pltpu.make_async_copy(vals_v.at[pl.ds(r * 128, 128)],
                              dst.at[dest_v.at[r]], sem).wait()
        return 0
      lax.fori_loop(0, nr, dloop, 0)

    d1 = lambda k: lax.bitwise_and(k, jnp.int32(511))
    pack1 = lambda k, j: lax.bitwise_or(
        lax.shift_left(lax.shift_right_logical(k, jnp.int32(9)),
                       jnp.int32(20)), j)
    pltpu.sync_copy(keys_hbm.at[pl.ds(base, cpt)], keys_v)
    zero_off()
    histogram(d1, pack1)
    tile_totals(b1)
    pltpu.sync_copy(tsum_v, grid_s.at[t])
    plsc.subcore_barrier()
    pltpu.sync_copy(grid_s, grid_v)
    compute_offsets(b1)
    rank_pass(d1)
    scatter_vals(p1_s)
    plsc.subcore_barrier()
    d2 = lambda p: lax.shift_right_logical(p, jnp.int32(20))
    unpack_j = lambda p, _: lax.bitwise_and(p, jnp.int32((1 << 20) - 1))
    pltpu.sync_copy(p1_s.at[pl.ds(base, cpt)], keys_v)
    zero_off()
    histogram(d2, unpack_j)
    tile_totals(256)
    pltpu.sync_copy(tsum_v, grid_s.at[t])
    plsc.subcore_barrier()
    pltpu.sync_copy(grid_s, grid_v)
    compute_offsets(256)
    rank_pass(d2)
    scatter_vals(ord_s)
    plsc.subcore_barrier()

    wid = c * NS + t
    wbase = wid * r_per_w
    pltpu.sync_copy(ord_s.at[pl.ds(wbase, r_per_w)], idx_v)

    def fire_gather(g, b):
      pltpu.async_copy(flat_hbm.at[idx_v.at[pl.ds(g * CH, CH)]], bufs[b],
                       gsems[b])

    def wait_gather(g, b):
      pltpu.make_async_copy(flat_hbm.at[idx_v.at[pl.ds(g * CH, CH)]], bufs[b],
                            gsems[b]).wait()

    def fire_write(g, b):
      pltpu.async_copy(bufs[b], out_hbm.at[pl.ds(wbase + g * CH, CH), :],
                       wsems[b])

    def wait_write(g, b):
      pltpu.make_async_copy(bufs[b], out_hbm.at[pl.ds(wbase + g * CH, CH), :],
                            wsems[b]).wait()

    fire_gather(0, 0)
    fire_gather(1, 1)
    for g in (0, 1):
      wait_gather(g, g)
      fire_write(g, g)
      fire_gather(g + 2, g + 2)

    def outer(j, _):
      for k in range(4):
        g = 2 + j * 4 + k
        b = (2 + k) % 4
        b2 = (b + 2) % 4
        wait_gather(g, b)
        fire_write(g, b)
        wait_write(g - 2, b2)
        fire_gather(g + 2, b2)
      return 0

    lax.fori_loop(0, (g_chunks - 4) // 4, outer, 0)
    for g in (g_chunks - 2, g_chunks - 1):
      wait_gather(g, g % 4)
      fire_write(g, g % 4)
    for g in range(g_chunks - 4, g_chunks):
      wait_write(g, g % 4)

  return densify_kernel(flat, keys)


def kernel(features, sample_idx):
  s, n_per, d = features.shape
  n = s * n_per
  flat = features.reshape(n, d)
  keys = sample_idx.reshape(n)
  return _densify(flat, keys)
